# Initial kernel scaffold; baseline (speedup 1.0000x reference)
import jax
import jax.numpy as jnp
from jax.experimental import pallas as pl


def kernel(b_t, codebook):
    raise NotImplementedError("write your pallas kernel here")


# speedup vs baseline: 1.4793x; 1.4793x over previous
import jax
import jax.numpy as jnp
from jax.experimental import pallas as pl

_N = 1024
_D = 64
_SIGMA = 0.1
_COMMIT_WEIGHT = 0.25
_BB = 256


def _body(z_ref, at_ref, st_ref, as_ref, asq_ref, l_ref, zsq_ref, vs_ref,
          ln_ref, cb_ref, zq_ref, idx_ref, lam_ref, dist_ref, loss_ref):
    z = z_ref[...]
    z_a = jnp.dot(z, at_ref[...], preferred_element_type=jnp.float32)
    z_s = jnp.dot(z, st_ref[...], preferred_element_type=jnp.float32)
    a_s = as_ref[...]
    a_sq = asq_ref[...]
    seg_len_sq = l_ref[...]
    z_sq = zsq_ref[...]

    t = (z_s - a_s) / seg_len_sq
    t = jnp.clip(t, 0.0, 1.0)
    d2 = z_sq - 2.0 * (z_a + t * z_s) + a_sq + 2.0 * t * a_s + t * t * seg_len_sq
    d2 = jnp.maximum(d2, 0.0)

    iota_i = jax.lax.broadcasted_iota(jnp.int32, (_BB, _N), 1)
    iota = iota_i.astype(jnp.float32)
    d2 = jnp.where(iota_i >= (_N - 1), jnp.float32(3.0e38), d2)

    m = jnp.min(d2, axis=1, keepdims=True)
    cand = jnp.where(d2 == m, iota, jnp.float32(_N))
    idxf = jnp.min(cand, axis=1, keepdims=True)

    onehot = iota == idxf
    onehot_n = iota == (idxf + 1.0)
    lam = jnp.sum(jnp.where(onehot, t, 0.0), axis=1, keepdims=True)
    lamd = jnp.clip(lam + ln_ref[...], 0.0, 1.0)
    w = jnp.where(onehot, 1.0 - lamd, 0.0) + jnp.where(onehot_n, lamd, 0.0)
    zq = jnp.dot(w, cb_ref[...], preferred_element_type=jnp.float32,
                 precision=jax.lax.Precision.HIGHEST) + vs_ref[...]

    zq_ref[...] = zq
    idx_ref[...] = idxf.astype(jnp.int32)
    lam_ref[...] = lam
    dist_ref[...] = jnp.sqrt(m)

    @pl.when(pl.program_id(0) == 0)
    def _init():
        loss_ref[...] = jnp.zeros_like(loss_ref)

    loss_ref[...] = loss_ref[...] + jnp.sum(m, axis=0, keepdims=True)


def kernel(b_t, codebook):
    if b_t.ndim == 1:
        b_t = b_t[None, :]
    b = b_t.shape[0]

    a = codebook[:-1]
    s = codebook[1:] - a
    seg_len_sq = jnp.sum(s * s, axis=-1) + 1e-8
    a_s = jnp.sum(a * s, axis=-1)
    a_sq = jnp.sum(a * a, axis=-1)

    zpad = jnp.zeros((1,), jnp.float32)
    l_p = jnp.concatenate([seg_len_sq, jnp.ones((1,), jnp.float32)])[None, :]
    as_p = jnp.concatenate([a_s, zpad])[None, :]
    asq_p = jnp.concatenate([a_sq, zpad])[None, :]
    zrow = jnp.zeros((1, _D), jnp.float32)
    at = jnp.concatenate([a, zrow], axis=0).T
    st = jnp.concatenate([s, zrow], axis=0).T

    z_sq = jnp.sum(b_t * b_t, axis=-1, keepdims=True)

    key = jax.random.key(42)
    k1, k2 = jax.random.split(key)
    v = jax.random.normal(k1, b_t.shape, dtype=b_t.dtype) * _SIGMA
    vs = v * _SIGMA * 0.1
    lam_noise = (jax.random.uniform(k2, (b,), dtype=b_t.dtype) * 0.1 - 0.05)[:, None]

    grid = (b // _BB,)
    row = lambda i: (i, 0)
    fixed = lambda i: (0, 0)
    zq, idx, lam, dist, loss_acc = pl.pallas_call(
        _body,
        grid=grid,
        in_specs=[
            pl.BlockSpec((_BB, _D), row),
            pl.BlockSpec((_D, _N), fixed),
            pl.BlockSpec((_D, _N), fixed),
            pl.BlockSpec((1, _N), fixed),
            pl.BlockSpec((1, _N), fixed),
            pl.BlockSpec((1, _N), fixed),
            pl.BlockSpec((_BB, 1), row),
            pl.BlockSpec((_BB, _D), row),
            pl.BlockSpec((_BB, 1), row),
            pl.BlockSpec((_N, _D), fixed),
        ],
        out_specs=[
            pl.BlockSpec((_BB, _D), row),
            pl.BlockSpec((_BB, 1), row),
            pl.BlockSpec((_BB, 1), row),
            pl.BlockSpec((_BB, 1), row),
            pl.BlockSpec((1, 1), fixed),
        ],
        out_shape=[
            jax.ShapeDtypeStruct((b, _D), jnp.float32),
            jax.ShapeDtypeStruct((b, 1), jnp.int32),
            jax.ShapeDtypeStruct((b, 1), jnp.float32),
            jax.ShapeDtypeStruct((b, 1), jnp.float32),
            jax.ShapeDtypeStruct((1, 1), jnp.float32),
        ],
    )(b_t, at, st, as_p, asq_p, l_p, z_sq, vs, lam_noise, codebook)

    loss = (loss_acc[0, 0] / (b * _D)) * _COMMIT_WEIGHT
    return zq, loss, idx[:, 0], lam[:, 0], dist[:, 0]
